# baseline (speedup 1.0000x reference)
import functools

import jax
import jax.numpy as jnp
from jax import lax
from jax.experimental import pallas as pl
from jax.experimental.pallas import tpu as pltpu
from jax.experimental.pallas import tpu_sc as plsc

TOTAL_TOKENS = 32768
BATCH = 16
D_MODEL = 4096
LANES = 16
HALF = D_MODEL // 2
N_HSLICES = HALF // LANES

_mesh = plsc.VectorSubcoreMesh(core_axis_name="c", subcore_axis_name="s",
                               num_cores=1)


def _ssq(ref):
    @functools.partial(
        plsc.parallel_loop(0, N_HSLICES, unroll=8,
                           carry=jnp.zeros((LANES,), jnp.float32))
    )
    def acc(k, a):
        x = ref[0, pl.ds(k * LANES, LANES)]
        return a + x * x

    return acc


def _scale(ref, y):
    @functools.partial(plsc.parallel_loop(0, N_HSLICES, unroll=8))
    def _(k):
        ref[0, pl.ds(k * LANES, LANES)] = ref[0, pl.ds(k * LANES, LANES)] * y


@functools.partial(
    pl.kernel,
    mesh=_mesh,
    out_type=jax.ShapeDtypeStruct((BATCH, D_MODEL), jnp.float32),
    scratch_types=[
        pltpu.VMEM((LANES,), jnp.int32),
        pltpu.VMEM((1,), jnp.int32),
        pltpu.VMEM((1, HALF), jnp.float32),
        pltpu.VMEM((1, HALF), jnp.float32),
        pltpu.SemaphoreType.DMA,
        pltpu.SemaphoreType.DMA,
    ],
    compiler_params=pltpu.CompilerParams(needs_layout_passes=False),
)
def _pooler(hs_hbm, lens_hbm, out_hbm, lens_v, idx1, row_a, row_b, sem_a, sem_b):
    r = lax.axis_index("s")

    pltpu.sync_copy(lens_hbm, lens_v)
    lens = lens_v[...]
    csum = jnp.cumsum(lens)
    iota = lax.iota(jnp.int32, LANES)
    plsc.store_scatter(idx1, [jnp.zeros((LANES,), jnp.int32)], csum - 1,
                       mask=iota == r)
    in_a = pltpu.async_copy(hs_hbm.at[idx1, pl.ds(0, HALF)], row_a, sem_a)
    in_b = pltpu.async_copy(hs_hbm.at[idx1, pl.ds(HALF, HALF)], row_b, sem_b)

    in_a.wait()
    acc = _ssq(row_a)
    in_b.wait()
    acc = acc + _ssq(row_b)

    ssv = jnp.full((LANES,), jnp.sum(acc))
    ssv = jnp.maximum(ssv, 1e-24)
    y = plsc.bitcast(0x5F3759DF - lax.shift_right_logical(
        plsc.bitcast(ssv, jnp.int32), 1), jnp.float32)
    for _unused in range(3):
        y = y * (1.5 - 0.5 * ssv * y * y)

    _scale(row_a, y)
    out_a = pltpu.async_copy(row_a, out_hbm.at[pl.ds(r, 1), pl.ds(0, HALF)],
                             sem_a)
    _scale(row_b, y)
    out_b = pltpu.async_copy(row_b, out_hbm.at[pl.ds(r, 1), pl.ds(HALF, HALF)],
                             sem_b)
    out_a.wait()
    out_b.wait()


def kernel(hidden_states, extend_seq_lens):
    return _pooler(hidden_states, extend_seq_lens)


# speedup vs baseline: 1.0065x; 1.0065x over previous
import functools

import jax
import jax.numpy as jnp
from jax import lax
from jax.experimental import pallas as pl
from jax.experimental.pallas import tpu as pltpu
from jax.experimental.pallas import tpu_sc as plsc

TOTAL_TOKENS = 32768
BATCH = 16
D_MODEL = 4096
LANES = 16
N_SLICES = D_MODEL // LANES

_mesh = plsc.VectorSubcoreMesh(core_axis_name="c", subcore_axis_name="s",
                               num_cores=1)


@functools.partial(
    pl.kernel,
    mesh=_mesh,
    out_type=jax.ShapeDtypeStruct((BATCH, D_MODEL), jnp.float32),
    scratch_types=[
        pltpu.VMEM((LANES,), jnp.int32),
        pltpu.VMEM((1,), jnp.int32),
        pltpu.VMEM((1, D_MODEL), jnp.float32),
        pltpu.SemaphoreType.DMA,
    ],
    compiler_params=pltpu.CompilerParams(needs_layout_passes=False),
)
def _pooler(hs_hbm, lens_hbm, out_hbm, lens_v, idx1, row_v, sem):
    r = lax.axis_index("s")

    pltpu.sync_copy(lens_hbm, lens_v)
    lens = lens_v[...]
    csum = jnp.cumsum(lens)
    iota = lax.iota(jnp.int32, LANES)
    plsc.store_scatter(idx1, [jnp.zeros((LANES,), jnp.int32)], csum - 1,
                       mask=iota == r)
    pltpu.async_copy(hs_hbm.at[idx1], row_v, sem).wait()

    @functools.partial(
        plsc.parallel_loop(0, N_SLICES, unroll=16,
                           carry=jnp.zeros((LANES,), jnp.float32))
    )
    def acc(k, a):
        x = row_v[0, pl.ds(k * LANES, LANES)]
        return a + x * x

    ssv = jnp.full((LANES,), jnp.sum(acc))
    ssv = jnp.maximum(ssv, 1e-24)
    y = plsc.bitcast(0x5F3759DF - lax.shift_right_logical(
        plsc.bitcast(ssv, jnp.int32), 1), jnp.float32)
    for _unused in range(3):
        y = y * (1.5 - 0.5 * ssv * y * y)

    @functools.partial(plsc.parallel_loop(0, N_SLICES, unroll=16))
    def _scale(k):
        row_v[0, pl.ds(k * LANES, LANES)] = row_v[0, pl.ds(k * LANES, LANES)] * y

    pltpu.sync_copy(row_v, out_hbm.at[pl.ds(r, 1)])


def kernel(hidden_states, extend_seq_lens):
    return _pooler(hidden_states, extend_seq_lens)


# speedup vs baseline: 1.0175x; 1.0110x over previous
import functools

import jax
import jax.numpy as jnp
from jax import lax
from jax.experimental import pallas as pl
from jax.experimental.pallas import tpu as pltpu
from jax.experimental.pallas import tpu_sc as plsc

TOTAL_TOKENS = 32768
BATCH = 16
D_MODEL = 4096
LANES = 16
N_SLICES = D_MODEL // LANES

_mesh = plsc.VectorSubcoreMesh(core_axis_name="c", subcore_axis_name="s",
                               num_cores=1)


@functools.partial(
    pl.kernel,
    mesh=_mesh,
    out_type=jax.ShapeDtypeStruct((BATCH, D_MODEL), jnp.float32),
    scratch_types=[
        pltpu.VMEM((LANES,), jnp.int32),
        pltpu.VMEM((1,), jnp.int32),
        pltpu.VMEM((1, D_MODEL), jnp.float32),
        pltpu.SemaphoreType.DMA,
    ],
    compiler_params=pltpu.CompilerParams(needs_layout_passes=False),
)
def _pooler(hs_hbm, lens_hbm, out_hbm, lens_v, idx1, row_v, sem):
    r = lax.axis_index("s")

    pltpu.sync_copy(lens_hbm, lens_v)
    lens = lens_v[...]
    csum = jnp.cumsum(lens)
    iota = lax.iota(jnp.int32, LANES)
    plsc.store_scatter(idx1, [jnp.zeros((LANES,), jnp.int32)], csum - 1,
                       mask=iota == r)
    pltpu.async_copy(hs_hbm.at[idx1], row_v, sem).wait()

    @functools.partial(
        plsc.parallel_loop(0, N_SLICES, unroll=8,
                           carry=jnp.zeros((LANES,), jnp.float32))
    )
    def acc(k, a):
        x = row_v[0, pl.ds(k * LANES, LANES)]
        return a + x * x

    ssv = jnp.full((LANES,), jnp.sum(acc))
    ssv = jnp.maximum(ssv, 1e-24)
    y = plsc.bitcast(0x5F3759DF - lax.shift_right_logical(
        plsc.bitcast(ssv, jnp.int32), 1), jnp.float32)
    for _unused in range(3):
        y = y * (1.5 - 0.5 * ssv * y * y)

    @functools.partial(plsc.parallel_loop(0, N_SLICES, unroll=8))
    def _scale(k):
        row_v[0, pl.ds(k * LANES, LANES)] = row_v[0, pl.ds(k * LANES, LANES)] * y

    pltpu.sync_copy(row_v, out_hbm.at[pl.ds(r, 1)])


def kernel(hidden_states, extend_seq_lens):
    return _pooler(hidden_states, extend_seq_lens)
